# baseline (speedup 1.0000x reference)
import functools

import jax
import jax.numpy as jnp
from jax import lax
from jax.experimental import pallas as pl
from jax.experimental.pallas import tpu as pltpu
from jax.experimental.pallas import tpu_sc as plsc

_PROMPT_LEN = 512
_EMBED_SIZE = 4096

_R_SC = 128
_ROWS_PER_W = 8
_ACTIVE_W = _R_SC // _ROWS_PER_W
_NC, _NS = 2, 16
_BLK = 64


@functools.partial(
    pl.kernel,
    mesh=plsc.VectorSubcoreMesh(core_axis_name="c", subcore_axis_name="s"),
    out_type=jax.ShapeDtypeStruct((_R_SC, _EMBED_SIZE), jnp.float32),
    scratch_types=[
        pltpu.VMEM((_ROWS_PER_W,), jnp.int32),
        pltpu.VMEM((_ROWS_PER_W, _EMBED_SIZE), jnp.float32),
        pltpu.SemaphoreType.DMA,
    ],
)
def _sc_gather(table_hbm, idx_hbm, out_hbm, idx_v, rows_v, sem):
    wid = lax.axis_index("s") * _NC + lax.axis_index("c")

    @pl.when(wid < _ACTIVE_W)
    def _():
        base = wid * _ROWS_PER_W
        pltpu.sync_copy(idx_hbm.at[pl.ds(base, _ROWS_PER_W)], idx_v)
        pltpu.async_copy(table_hbm.at[idx_v], rows_v, sem).wait()
        pltpu.sync_copy(rows_v, out_hbm.at[pl.ds(base, _ROWS_PER_W)])


def _tc_body(in_ref, out_ref):
    out_ref[...] = in_ref[...]


def _tc_copy_tail(table):
    n_blk = (_PROMPT_LEN - _R_SC) // _BLK
    off = _R_SC // _BLK
    return pl.pallas_call(
        _tc_body,
        grid=(n_blk,),
        in_specs=[pl.BlockSpec((_BLK, _EMBED_SIZE), lambda i: (i + off, 0))],
        out_specs=pl.BlockSpec((_BLK, _EMBED_SIZE), lambda i: (i + off, 0)),
        out_shape=jax.ShapeDtypeStruct((_PROMPT_LEN, _EMBED_SIZE), jnp.float32),
    )(table)


def kernel(prompt_table, indices):
    sc_rows = _sc_gather(prompt_table, indices[:_R_SC])
    tc_full = _tc_copy_tail(prompt_table)
    return lax.dynamic_update_slice(tc_full, sc_rows, (0, 0))


# speedup vs baseline: 1.0743x; 1.0743x over previous
import functools

import jax
import jax.numpy as jnp
from jax import lax
from jax.experimental import pallas as pl
from jax.experimental.pallas import tpu as pltpu
from jax.experimental.pallas import tpu_sc as plsc

_PROMPT_LEN = 512
_EMBED_SIZE = 4096

_NC, _NS = 2, 16
_NW = _NC * _NS
_ROWS_PER_W = _PROMPT_LEN // _NW
_CH = 8
_NCH = _ROWS_PER_W // _CH


@functools.partial(
    pl.kernel,
    mesh=plsc.VectorSubcoreMesh(core_axis_name="c", subcore_axis_name="s"),
    out_type=jax.ShapeDtypeStruct((_PROMPT_LEN, _EMBED_SIZE), jnp.float32),
    scratch_types=[
        pltpu.VMEM((_ROWS_PER_W,), jnp.int32),
        pltpu.VMEM((2, _CH, _EMBED_SIZE), jnp.float32),
        pltpu.SemaphoreType.DMA,
        pltpu.SemaphoreType.DMA,
    ],
)
def _gather_rows(table_hbm, idx_hbm, out_hbm, idx_v, buf, gsem, ssem):
    wid = lax.axis_index("s") * _NC + lax.axis_index("c")
    base = wid * _ROWS_PER_W
    pltpu.sync_copy(idx_hbm.at[pl.ds(base, _ROWS_PER_W)], idx_v)
    gathers = [
        pltpu.make_async_copy(
            table_hbm.at[idx_v.at[pl.ds(c * _CH, _CH)]], buf.at[c % 2], gsem
        )
        for c in range(_NCH)
    ]
    stores = [
        pltpu.make_async_copy(
            buf.at[c % 2], out_hbm.at[pl.ds(base + c * _CH, _CH)], ssem
        )
        for c in range(_NCH)
    ]
    gathers[0].start()
    for c in range(_NCH):
        if c + 1 < _NCH:
            if c >= 1:
                stores[c - 1].wait()
            gathers[c + 1].start()
        gathers[c].wait()
        stores[c].start()
    stores[_NCH - 2].wait()
    stores[_NCH - 1].wait()


def kernel(prompt_table, indices):
    return _gather_rows(prompt_table, indices)
